# Initial kernel scaffold; baseline (speedup 1.0000x reference)
import jax
import jax.numpy as jnp
from jax.experimental import pallas as pl

HEADS = 4


def kernel(verts, params, edges):
    x = verts
    N = x.shape[0]
    src, dst = edges[0], edges[1]
    deg = jax.ops.segment_sum(jnp.ones((src.shape[0],), jnp.float32), dst, num_segments=N) + 1.0
    conv_outputs = []
    for p in params['convs']:
        d_out = p['W'].shape[1] // HEADS
        xW = x @ p['W']
        xU = x @ p['U']
        lo = xU[src] - xU[dst] + p['c']
        q = jax.nn.softmax(lo, axis=1)
        xWh = xW.reshape(N, HEADS, d_out)
        msg = jnp.einsum('eh,ehd->ed', q, xWh[src])
        agg = jax.ops.segment_sum(msg, dst, num_segments=N)
        q0 = jax.nn.softmax(p['c'])
        agg = agg + jnp.einsum('h,nhd->nd', q0, xWh)
        x = jax.nn.elu(agg / deg[:, None] + p['b'])
        conv_outputs.append(x)
    xc = jnp.concatenate(conv_outputs, axis=1)
    y = params['conv1d_w'] @ xc.T + params['conv1d_b'][:, None]
    y = jnp.where(y >= 0, y, 0.2 * y)
    x1 = jnp.max(y, axis=1).reshape(1, -1)
    x2 = jnp.mean(y, axis=1).reshape(1, -1)
    z = jnp.concatenate([x1, x2], axis=1)
    for (W, b) in params['lins'][:-1]:
        z = jax.nn.elu(z @ W + b)
    W, b = params['lins'][-1]
    return jnp.tanh(z @ W + b)


# speedup vs baseline: 2.8775x; 2.8775x over previous
import functools
import jax
import jax.numpy as jnp
from jax import lax
from jax.experimental import pallas as pl
from jax.experimental.pallas import tpu as pltpu
from jax.experimental.pallas import tpu_sc as plsc

H = 4
N = 10000
C = 128
NPAD = 10240
NB = 256
HB = 400
NEG = -1e30


def _mm_body(x_ref, w_ref, u_ref, xw_ref, xu_ref):
    x = x_ref[...]
    xw_ref[...] = jnp.dot(x, w_ref[...], preferred_element_type=jnp.float32)
    xu_ref[...] = jnp.dot(x, u_ref[...], preferred_element_type=jnp.float32)


def _mm(x, W640, U128):
    nblk = NPAD // NB
    return pl.pallas_call(
        _mm_body,
        grid=(nblk,),
        in_specs=[pl.BlockSpec((NB, C), lambda i: (i, 0)),
                  pl.BlockSpec((C, 5 * C), lambda i: (0, 0)),
                  pl.BlockSpec((C, C), lambda i: (0, 0))],
        out_specs=[pl.BlockSpec((NB, 5 * C), lambda i: (i, 0)),
                   pl.BlockSpec((NB, C), lambda i: (i, 0))],
        out_shape=[jax.ShapeDtypeStruct((NPAD, 5 * C), jnp.float32),
                   jax.ShapeDtypeStruct((NPAD, C), jnp.float32)],
    )(x, W640, U128)


def _epi_body(agg_ref, xw_ref, deg_ref, cpad_ref, b_ref, out_ref):
    a = agg_ref[0] + agg_ref[1]
    deg = deg_ref[0] + deg_ref[1] + 1.0
    cp = cpad_ref[...]
    m = jnp.max(cp, axis=1, keepdims=True)
    e = jnp.exp(cp - m)
    q0 = e / jnp.sum(e, axis=1, keepdims=True)
    xw = xw_ref[...]
    sm = jnp.zeros_like(a)
    for h in range(H):
        qh = q0[0:1, h:h + 1]
        sm = sm + qh * xw[:, h * C:(h + 1) * C]
    y = (a + sm) / deg + b_ref[...]
    out_ref[...] = jnp.where(y > 0, y, jnp.exp(y) - 1.0)


def _epilogue(agg2, xw, deg2, cpad, brow):
    nblk = NPAD // NB
    return pl.pallas_call(
        _epi_body,
        grid=(nblk,),
        in_specs=[pl.BlockSpec((2, NB, C), lambda i: (0, i, 0)),
                  pl.BlockSpec((NB, 5 * C), lambda i: (i, 0)),
                  pl.BlockSpec((2, NB, C), lambda i: (0, i, 0)),
                  pl.BlockSpec((1, C), lambda i: (0, 0)),
                  pl.BlockSpec((1, C), lambda i: (0, 0))],
        out_specs=pl.BlockSpec((NB, C), lambda i: (i, 0)),
        out_shape=jax.ShapeDtypeStruct((NPAD, C), jnp.float32),
    )(agg2, xw, deg2, cpad, brow)


def _head_body(x1_ref, x2_ref, x3_ref, wc1, wc2, wc3, cb_ref,
               w0a, w0b, b0, w1, b1, w2, b2, out_ref, maxs, sums):
    i = pl.program_id(0)
    nblk = pl.num_programs(0)
    dn = (((1,), (1,)), ((), ()))
    y = lax.dot_general(wc1[...], x1_ref[...], dn, preferred_element_type=jnp.float32)
    y = y + lax.dot_general(wc2[...], x2_ref[...], dn, preferred_element_type=jnp.float32)
    y = y + lax.dot_general(wc3[...], x3_ref[...], dn, preferred_element_type=jnp.float32)
    y = y + cb_ref[:, 0:1]
    y = jnp.where(y >= 0, y, 0.2 * y)
    ymax = jnp.broadcast_to(jnp.max(y, axis=1, keepdims=True), (8 * C, C))
    ysum = jnp.broadcast_to(jnp.sum(y, axis=1, keepdims=True), (8 * C, C))

    @pl.when(i == 0)
    def _():
        maxs[...] = ymax
        sums[...] = ysum

    @pl.when(i > 0)
    def _():
        maxs[...] = jnp.maximum(maxs[...], ymax)
        sums[...] = sums[...] + ysum

    @pl.when(i == nblk - 1)
    def _():
        dc = (((0,), (0,)), ((), ()))
        z = lax.dot_general(maxs[...], w0a[...], dc, preferred_element_type=jnp.float32)
        z = z + lax.dot_general(sums[...] * (1.0 / N), w0b[...], dc,
                                preferred_element_type=jnp.float32)
        z = z + b0[...]
        z = jnp.where(z > 0, z, jnp.exp(z) - 1.0)
        z = jnp.dot(z, w1[...], preferred_element_type=jnp.float32) + b1[...]
        z = jnp.where(z > 0, z, jnp.exp(z) - 1.0)
        z = jnp.dot(z, w2[...], preferred_element_type=jnp.float32) + b2[...]
        out_ref[...] = jnp.tanh(z[0:1, :])


def _head(x1, x2, x3, wc1, wc2, wc3, cb8, w0a, w0b, b0, w1, b1, w2p, b2p):
    nblk = N // HB
    return pl.pallas_call(
        _head_body,
        grid=(nblk,),
        in_specs=[pl.BlockSpec((HB, C), lambda i: (i, 0)),
                  pl.BlockSpec((HB, C), lambda i: (i, 0)),
                  pl.BlockSpec((HB, C), lambda i: (i, 0)),
                  pl.BlockSpec((8 * C, C), lambda i: (0, 0)),
                  pl.BlockSpec((8 * C, C), lambda i: (0, 0)),
                  pl.BlockSpec((8 * C, C), lambda i: (0, 0)),
                  pl.BlockSpec((8 * C, 8), lambda i: (0, 0)),
                  pl.BlockSpec((8 * C, 512), lambda i: (0, 0)),
                  pl.BlockSpec((8 * C, 512), lambda i: (0, 0)),
                  pl.BlockSpec((1, 512), lambda i: (0, 0)),
                  pl.BlockSpec((512, 256), lambda i: (0, 0)),
                  pl.BlockSpec((1, 256), lambda i: (0, 0)),
                  pl.BlockSpec((256, C), lambda i: (0, 0)),
                  pl.BlockSpec((1, C), lambda i: (0, 0))],
        out_specs=pl.BlockSpec((1, C), lambda i: (0, 0)),
        out_shape=jax.ShapeDtypeStruct((1, C), jnp.float32),
        scratch_shapes=[pltpu.VMEM((8 * C, C), jnp.float32),
                        pltpu.VMEM((8 * C, C), jnp.float32)],
    )(x1, x2, x3, wc1, wc2, wc3, cb8, w0a, w0b, b0, w1, b1, w2p, b2p)


EP = 163840
EPT = EP // 32
CH = 32
NROW = NPAD // 16


def _lane_gather(v, idx):
    dn = lax.GatherDimensionNumbers(offset_dims=(), collapsed_slice_dims=(0,),
                                    start_index_map=(0,))
    return lax.gather(v, idx[:, None], dn, (1,),
                      mode=lax.GatherScatterMode.PROMISE_IN_BOUNDS)


def _lane_bcast(v, t):
    return _lane_gather(v, jnp.full((16,), t, jnp.int32))


def _edge_body(xw, xu, srcr, dstr, cvec, zer, out,
               agg_sh, cvec_v, idx_s, idx_d, rows_v, ud_v, msg_v,
               sem0, sem1):
    cid = lax.axis_index("c")
    sid = lax.axis_index("s")
    wid = sid * 2 + cid
    pltpu.sync_copy(zer, agg_sh.at[pl.ds(sid * NROW, NROW)])
    pltpu.sync_copy(cvec, cvec_v)
    plsc.subcore_barrier()

    lane = lax.iota(jnp.int32, 16)
    rot1 = jnp.bitwise_or(jnp.bitwise_and(lane, 12),
                          jnp.bitwise_and(lane + 1, 3))
    rot2 = jnp.bitwise_or(jnp.bitwise_and(lane, 12),
                          jnp.bitwise_and(lane + 2, 3))

    def chunk(g, carry):
        base = wid * EPT + g * CH
        pltpu.sync_copy(srcr.at[pl.ds(base, CH)], idx_s)
        pltpu.sync_copy(dstr.at[pl.ds(base, CH)], idx_d)
        cp0 = pltpu.async_copy(xw.at[idx_s], rows_v, sem0)
        cp1 = pltpu.async_copy(xu.at[idx_d], ud_v, sem1)
        cv = cvec_v[...]
        cp1.wait()
        cp0.wait()
        for j in range(CH // 16):
            for t in range(16):
                e = j * 16 + t
                l = rows_v[e, pl.ds(4 * C, 16)] - ud_v[e, pl.ds(0, 16)] + cv
                m = jnp.maximum(l, _lane_gather(l, rot1))
                m = jnp.maximum(m, _lane_gather(m, rot2))
                ex = jnp.exp(l - m)
                sm = ex + _lane_gather(ex, rot1)
                sm = sm + _lane_gather(sm, rot2)
                q = ex / sm
                qs = [_lane_bcast(q, h) for h in range(H)]
                for k in range(C // 16):
                    acc = qs[0] * rows_v[e, pl.ds(k * 16, 16)]
                    acc = acc + qs[1] * rows_v[e, pl.ds(C + k * 16, 16)]
                    acc = acc + qs[2] * rows_v[e, pl.ds(2 * C + k * 16, 16)]
                    acc = acc + qs[3] * rows_v[e, pl.ds(3 * C + k * 16, 16)]
                    msg_v[e, pl.ds(k * 16, 16)] = acc
        pltpu.sync_copy(msg_v, agg_sh.at[idx_d], add=True)
        return carry

    lax.fori_loop(0, EPT // CH, chunk, 0)
    plsc.subcore_barrier()
    pltpu.sync_copy(agg_sh.at[pl.ds(sid * NROW, NROW)],
                    out.at[cid, pl.ds(sid * NROW, NROW)])


@functools.partial(
    pl.kernel,
    out_type=jax.ShapeDtypeStruct((2, NPAD, C), jnp.float32),
    mesh=plsc.VectorSubcoreMesh(core_axis_name="c", subcore_axis_name="s"),
    compiler_params=pltpu.CompilerParams(needs_layout_passes=False),
    scratch_types=[
        pltpu.VMEM_SHARED((NPAD, C), jnp.float32),
        pltpu.VMEM((16,), jnp.float32),
        pltpu.VMEM((CH,), jnp.int32),
        pltpu.VMEM((CH,), jnp.int32),
        pltpu.VMEM((CH, 5 * C), jnp.float32),
        pltpu.VMEM((CH, C), jnp.float32),
        pltpu.VMEM((CH, C), jnp.float32),
        pltpu.SemaphoreType.DMA,
        pltpu.SemaphoreType.DMA,
    ],
)
def _sc_edge(xw, xu, srcr, dstr, cvec, zer, out, *rest):
    _edge_body(xw, xu, srcr, dstr, cvec, zer, out, *rest)


def _deg_body(dstr, onesr, zer, out, deg_sh, idx_d, ones_v, sem):
    cid = lax.axis_index("c")
    sid = lax.axis_index("s")
    wid = sid * 2 + cid
    pltpu.sync_copy(zer, deg_sh.at[pl.ds(sid * NROW, NROW)])
    pltpu.sync_copy(onesr, ones_v)
    plsc.subcore_barrier()

    def chunk(g, carry):
        base = wid * EPT + g * CH
        pltpu.sync_copy(dstr.at[pl.ds(base, CH)], idx_d)
        pltpu.sync_copy(ones_v, deg_sh.at[idx_d], add=True)
        return carry

    lax.fori_loop(0, EPT // CH, chunk, 0)
    plsc.subcore_barrier()
    pltpu.sync_copy(deg_sh.at[pl.ds(sid * NROW, NROW)],
                    out.at[cid, pl.ds(sid * NROW, NROW)])


@functools.partial(
    pl.kernel,
    out_type=jax.ShapeDtypeStruct((2, NPAD, C), jnp.float32),
    mesh=plsc.VectorSubcoreMesh(core_axis_name="c", subcore_axis_name="s"),
    compiler_params=pltpu.CompilerParams(needs_layout_passes=False),
    scratch_types=[
        pltpu.VMEM_SHARED((NPAD, C), jnp.float32),
        pltpu.VMEM((CH,), jnp.int32),
        pltpu.VMEM((CH, C), jnp.float32),
        pltpu.SemaphoreType.DMA,
    ],
)
def _sc_deg(dstr, onesr, zer, out, *rest):
    _deg_body(dstr, onesr, zer, out, *rest)


def kernel(verts, params, edges):
    src, dst = edges[0], edges[1]
    E = src.shape[0]
    x = jnp.pad(verts, ((0, NPAD - N), (0, 0)))

    srcp = jnp.concatenate([src, jnp.zeros((EP - E,), src.dtype)])
    dstp = jnp.concatenate([dst, jnp.full((EP - E,), N, dst.dtype)])

    zer128 = jnp.zeros((NROW, C), jnp.float32)
    ones128 = jnp.ones((CH, C), jnp.float32)

    deg2 = _sc_deg(dstp, ones128, zer128)

    xs = []
    for p in params['convs']:
        U16 = jnp.tile(p['U'], (1, 4))
        W640 = jnp.concatenate(
            [p['W'], U16, jnp.zeros((C, C - 16), jnp.float32)], axis=1)
        U128 = jnp.concatenate(
            [U16, jnp.zeros((C, C - 16), jnp.float32)], axis=1)
        cpad = jnp.full((1, C), NEG, jnp.float32).at[0, :H].set(p['c'])
        cvec = jnp.tile(p['c'], 4)
        brow = p['b'].reshape(1, C)
        xw, xu = _mm(x, W640, U128)
        agg2 = _sc_edge(xw, xu, srcp, dstp, cvec, zer128)
        x = _epilogue(agg2, xw, deg2, cpad, brow)
        xs.append(x[:N])

    wc = params['conv1d_w']
    cb8 = jnp.broadcast_to(params['conv1d_b'][:, None], (8 * C, 8))
    (W0, b0), (W1, b1), (W2, b2) = params['lins']
    w2p = jnp.pad(W2, ((0, 0), (0, C - W2.shape[1])))
    b2p = jnp.pad(b2, (0, C - b2.shape[0])).reshape(1, C)
    out = _head(xs[0], xs[1], xs[2],
                wc[:, 0:C], wc[:, C:2 * C], wc[:, 2 * C:3 * C], cb8,
                W0[:8 * C], W0[8 * C:], b0.reshape(1, 512),
                W1, b1.reshape(1, 256), w2p, b2p)
    return out[:, :10]


# speedup vs baseline: 3.2574x; 1.1321x over previous
import functools
import jax
import jax.numpy as jnp
from jax import lax
from jax.experimental import pallas as pl
from jax.experimental.pallas import tpu as pltpu
from jax.experimental.pallas import tpu_sc as plsc

H = 4
N = 10000
C = 128
NPAD = 10112
NB = 128
HB = 400
NEG = -1e30


def _mm_body(x_ref, w_ref, u_ref, b_ref, xw_ref, xu_ref):
    x = x_ref[...]
    xw_ref[...] = jnp.dot(x, w_ref[...],
                          preferred_element_type=jnp.float32) + b_ref[...]
    xu_ref[...] = jnp.dot(x, u_ref[...], preferred_element_type=jnp.float32)


def _mm(x, W640, U128, b640):
    nblk = NPAD // NB
    return pl.pallas_call(
        _mm_body,
        grid=(nblk,),
        in_specs=[pl.BlockSpec((NB, C), lambda i: (i, 0)),
                  pl.BlockSpec((C, 5 * C), lambda i: (0, 0)),
                  pl.BlockSpec((C, C), lambda i: (0, 0)),
                  pl.BlockSpec((1, 5 * C), lambda i: (0, 0))],
        out_specs=[pl.BlockSpec((NB, 5 * C), lambda i: (i, 0)),
                   pl.BlockSpec((NB, C), lambda i: (i, 0))],
        out_shape=[jax.ShapeDtypeStruct((NPAD, 5 * C), jnp.float32),
                   jax.ShapeDtypeStruct((NPAD, C), jnp.float32)],
    )(x, W640, U128, b640)


def _epi_body(agg_ref, xw_ref, deg_ref, cpad_ref, b_ref, out_ref):
    a = agg_ref[0] + agg_ref[1]
    deg = deg_ref[0] + deg_ref[1] + 1.0
    cp = cpad_ref[...]
    m = jnp.max(cp, axis=1, keepdims=True)
    e = jnp.exp(cp - m)
    q0 = e / jnp.sum(e, axis=1, keepdims=True)
    xw = xw_ref[...]
    sm = jnp.zeros_like(a)
    for h in range(H):
        qh = q0[0:1, h:h + 1]
        sm = sm + qh * xw[:, h * C:(h + 1) * C]
    y = (a + sm) / deg + b_ref[...]
    out_ref[...] = jnp.where(y > 0, y, jnp.exp(y) - 1.0)


def _epilogue(agg2, xw, deg2, cpad, brow):
    nblk = NPAD // NB
    return pl.pallas_call(
        _epi_body,
        grid=(nblk,),
        in_specs=[pl.BlockSpec((2, NB, C), lambda i: (0, i, 0)),
                  pl.BlockSpec((NB, 5 * C), lambda i: (i, 0)),
                  pl.BlockSpec((2, NB, C), lambda i: (0, i, 0)),
                  pl.BlockSpec((1, C), lambda i: (0, 0)),
                  pl.BlockSpec((1, C), lambda i: (0, 0))],
        out_specs=pl.BlockSpec((NB, C), lambda i: (i, 0)),
        out_shape=jax.ShapeDtypeStruct((NPAD, C), jnp.float32),
    )(agg2, xw, deg2, cpad, brow)


def _head_body(x1_ref, x2_ref, x3_ref, wc1, wc2, wc3, cb_ref,
               w0a, w0b, b0, w1, b1, w2, b2, out_ref, maxs, sums):
    i = pl.program_id(0)
    nblk = pl.num_programs(0)
    dn = (((1,), (1,)), ((), ()))
    y = lax.dot_general(wc1[...], x1_ref[...], dn, preferred_element_type=jnp.float32)
    y = y + lax.dot_general(wc2[...], x2_ref[...], dn, preferred_element_type=jnp.float32)
    y = y + lax.dot_general(wc3[...], x3_ref[...], dn, preferred_element_type=jnp.float32)
    y = y + cb_ref[:, 0:1]
    y = jnp.where(y >= 0, y, 0.2 * y)
    ymax = jnp.broadcast_to(jnp.max(y, axis=1, keepdims=True), (8 * C, C))
    ysum = jnp.broadcast_to(jnp.sum(y, axis=1, keepdims=True), (8 * C, C))

    @pl.when(i == 0)
    def _():
        maxs[...] = ymax
        sums[...] = ysum

    @pl.when(i > 0)
    def _():
        maxs[...] = jnp.maximum(maxs[...], ymax)
        sums[...] = sums[...] + ysum

    @pl.when(i == nblk - 1)
    def _():
        dc = (((0,), (0,)), ((), ()))
        z = lax.dot_general(maxs[...], w0a[...], dc, preferred_element_type=jnp.float32)
        z = z + lax.dot_general(sums[...] * (1.0 / N), w0b[...], dc,
                                preferred_element_type=jnp.float32)
        z = z + b0[...]
        z = jnp.where(z > 0, z, jnp.exp(z) - 1.0)
        z = jnp.dot(z, w1[...], preferred_element_type=jnp.float32) + b1[...]
        z = jnp.where(z > 0, z, jnp.exp(z) - 1.0)
        z = jnp.dot(z, w2[...], preferred_element_type=jnp.float32) + b2[...]
        out_ref[...] = jnp.tanh(z[0:1, :])


def _head(x1, x2, x3, wc1, wc2, wc3, cb8, w0a, w0b, b0, w1, b1, w2p, b2p):
    nblk = N // HB
    return pl.pallas_call(
        _head_body,
        grid=(nblk,),
        in_specs=[pl.BlockSpec((HB, C), lambda i: (i, 0)),
                  pl.BlockSpec((HB, C), lambda i: (i, 0)),
                  pl.BlockSpec((HB, C), lambda i: (i, 0)),
                  pl.BlockSpec((8 * C, C), lambda i: (0, 0)),
                  pl.BlockSpec((8 * C, C), lambda i: (0, 0)),
                  pl.BlockSpec((8 * C, C), lambda i: (0, 0)),
                  pl.BlockSpec((8 * C, 8), lambda i: (0, 0)),
                  pl.BlockSpec((8 * C, 512), lambda i: (0, 0)),
                  pl.BlockSpec((8 * C, 512), lambda i: (0, 0)),
                  pl.BlockSpec((1, 512), lambda i: (0, 0)),
                  pl.BlockSpec((512, 256), lambda i: (0, 0)),
                  pl.BlockSpec((1, 256), lambda i: (0, 0)),
                  pl.BlockSpec((256, C), lambda i: (0, 0)),
                  pl.BlockSpec((1, C), lambda i: (0, 0))],
        out_specs=pl.BlockSpec((1, C), lambda i: (0, 0)),
        out_shape=jax.ShapeDtypeStruct((1, C), jnp.float32),
        scratch_shapes=[pltpu.VMEM((8 * C, C), jnp.float32),
                        pltpu.VMEM((8 * C, C), jnp.float32)],
    )(x1, x2, x3, wc1, wc2, wc3, cb8, w0a, w0b, b0, w1, b1, w2p, b2p)


EP = 163840
EPT = EP // 32
CH = 32
AGGR = 10112
AGGROW = AGGR // 16
NROW = NPAD // 16


def _lane_gather(v, idx):
    dn = lax.GatherDimensionNumbers(offset_dims=(), collapsed_slice_dims=(0,),
                                    start_index_map=(0,))
    return lax.gather(v, idx[:, None], dn, (1,),
                      mode=lax.GatherScatterMode.PROMISE_IN_BOUNDS)


def _lane_bcast(v, t):
    return _lane_gather(v, jnp.full((16,), t, jnp.int32))


def _edge_body(xw, xu, srcr, dstr, zer, out,
               agg_sh, ixs0, ixs1, ixd0, ixd1, ixd2, ixd3,
               rows0, rows1, ud_v, msg_v, semr0, semr1, semu, semw, semi):
    cid = lax.axis_index("c")
    sid = lax.axis_index("s")
    wid = sid * 2 + cid
    pltpu.sync_copy(zer, agg_sh.at[pl.ds(sid * AGGROW, AGGROW)])
    plsc.subcore_barrier()

    lane = lax.iota(jnp.int32, 16)
    rot1 = jnp.bitwise_or(jnp.bitwise_and(lane, 12),
                          jnp.bitwise_and(lane + 1, 3))
    rot2 = jnp.bitwise_or(jnp.bitwise_and(lane, 12),
                          jnp.bitwise_and(lane + 2, 3))
    ixs = (ixs0, ixs1)
    ixd = (ixd0, ixd1, ixd2, ixd3)
    rows = (rows0, rows1)
    semr = (semr0, semr1)
    NCH = EPT // CH
    srow = wid * NCH

    def fire_idx(g, s2, s4):
        pltpu.async_copy(srcr.at[g], ixs[s2], semi)
        pltpu.async_copy(dstr.at[g], ixd[s4], semi)

    def wait_idx(s2, s4):
        pltpu.make_async_copy(srcr.at[0], ixs[s2], semi).wait()
        pltpu.make_async_copy(dstr.at[0], ixd[s4], semi).wait()

    fire_idx(srow, 0, 0)
    wait_idx(0, 0)
    pltpu.async_copy(xw.at[ixs0], rows0, semr0)
    pltpu.async_copy(xu.at[ixd0], ud_v, semu)
    fire_idx(srow + 1, 1, 1)
    fire_idx(srow + 2, 0, 2)

    def quad(Q, carry):
        for r in range(4):
            g = Q * 4 + r
            b = r & 1
            rv = rows[b]
            rn = (r + 1) & 3
            if r > 0:
                pltpu.make_async_copy(
                    msg_v, agg_sh.at[ixd[r - 1]], semw).wait()
            else:
                @pl.when(Q > 0)
                def _():
                    pltpu.make_async_copy(
                        msg_v, agg_sh.at[ixd[3]], semw).wait()

            @pl.when(g + 1 < NCH)
            def _():
                wait_idx((r + 1) & 1, rn)
                pltpu.async_copy(xw.at[ixs[(r + 1) & 1]], rows[1 - b],
                                 semr[1 - b])

            @pl.when(g + 3 < NCH)
            def _():
                fire_idx(srow + g + 3, (r + 3) & 1, (r + 3) & 3)

            pltpu.make_async_copy(xw.at[ixs[b]], rv, semr[b]).wait()
            pltpu.make_async_copy(xu.at[ixd[r]], ud_v, semu).wait()

            def edge(e, c3):
                l = rv[e, pl.ds(4 * C, 16)] - ud_v[e, pl.ds(0, 16)]
                m = jnp.maximum(l, _lane_gather(l, rot1))
                m = jnp.maximum(m, _lane_gather(m, rot2))
                ex = jnp.exp(l - m)
                sm = ex + _lane_gather(ex, rot1)
                sm = sm + _lane_gather(sm, rot2)
                q = ex / sm
                qs = [_lane_bcast(q, h) for h in range(H)]
                for k in range(C // 16):
                    acc = qs[0] * rv[e, pl.ds(k * 16, 16)]
                    acc = acc + qs[1] * rv[e, pl.ds(C + k * 16, 16)]
                    acc = acc + qs[2] * rv[e, pl.ds(2 * C + k * 16, 16)]
                    acc = acc + qs[3] * rv[e, pl.ds(3 * C + k * 16, 16)]
                    msg_v[e, pl.ds(k * 16, 16)] = acc
                return c3

            lax.fori_loop(0, CH, edge, 0, unroll=4)

            @pl.when(g + 1 < NCH)
            def _():
                pltpu.async_copy(xu.at[ixd[rn]], ud_v, semu)

            pltpu.async_copy(msg_v, agg_sh.at[ixd[r]], semw, add=True)
        return carry

    lax.fori_loop(0, NCH // 4, quad, 0)
    pltpu.make_async_copy(msg_v, agg_sh.at[ixd[3]], semw).wait()
    plsc.subcore_barrier()
    pltpu.sync_copy(agg_sh.at[pl.ds(sid * AGGROW, AGGROW)],
                    out.at[cid, pl.ds(sid * AGGROW, AGGROW)])


@functools.partial(
    pl.kernel,
    out_type=jax.ShapeDtypeStruct((2, NPAD, C), jnp.float32),
    mesh=plsc.VectorSubcoreMesh(core_axis_name="c", subcore_axis_name="s"),
    compiler_params=pltpu.CompilerParams(needs_layout_passes=False),
    scratch_types=[
        pltpu.VMEM_SHARED((AGGR, C), jnp.float32),
        pltpu.VMEM((CH,), jnp.int32),
        pltpu.VMEM((CH,), jnp.int32),
        pltpu.VMEM((CH,), jnp.int32),
        pltpu.VMEM((CH,), jnp.int32),
        pltpu.VMEM((CH,), jnp.int32),
        pltpu.VMEM((CH,), jnp.int32),
        pltpu.VMEM((CH, 5 * C), jnp.float32),
        pltpu.VMEM((CH, 5 * C), jnp.float32),
        pltpu.VMEM((CH, C), jnp.float32),
        pltpu.VMEM((CH, C), jnp.float32),
        pltpu.SemaphoreType.DMA,
        pltpu.SemaphoreType.DMA,
        pltpu.SemaphoreType.DMA,
        pltpu.SemaphoreType.DMA,
        pltpu.SemaphoreType.DMA,
    ],
)
def _sc_edge(xw, xu, srcr, dstr, zer, out, *rest):
    _edge_body(xw, xu, srcr, dstr, zer, out, *rest)


def _deg_body(dstr, onesr, zer, out, deg_sh, idx_d, ones_v, sem):
    cid = lax.axis_index("c")
    sid = lax.axis_index("s")
    wid = sid * 2 + cid
    pltpu.sync_copy(zer, deg_sh.at[pl.ds(sid * AGGROW, AGGROW)])
    pltpu.sync_copy(onesr, ones_v)
    plsc.subcore_barrier()

    def chunk(g, carry):
        base = wid * (EPT // CH) + g
        pltpu.sync_copy(dstr.at[base], idx_d)
        pltpu.sync_copy(ones_v, deg_sh.at[idx_d], add=True)
        return carry

    lax.fori_loop(0, EPT // CH, chunk, 0)
    plsc.subcore_barrier()
    pltpu.sync_copy(deg_sh.at[pl.ds(sid * AGGROW, AGGROW)],
                    out.at[cid, pl.ds(sid * AGGROW, AGGROW)])


@functools.partial(
    pl.kernel,
    out_type=jax.ShapeDtypeStruct((2, NPAD, C), jnp.float32),
    mesh=plsc.VectorSubcoreMesh(core_axis_name="c", subcore_axis_name="s"),
    compiler_params=pltpu.CompilerParams(needs_layout_passes=False),
    scratch_types=[
        pltpu.VMEM_SHARED((AGGR, C), jnp.float32),
        pltpu.VMEM((CH,), jnp.int32),
        pltpu.VMEM((CH, C), jnp.float32),
        pltpu.SemaphoreType.DMA,
    ],
)
def _sc_deg(dstr, onesr, zer, out, *rest):
    _deg_body(dstr, onesr, zer, out, *rest)


def kernel(verts, params, edges):
    src, dst = edges[0], edges[1]
    E = src.shape[0]
    x = jnp.pad(verts, ((0, NPAD - N), (0, 0)))

    srcp = jnp.concatenate([src, jnp.zeros((EP - E,), src.dtype)]).reshape(
        EP // CH, CH)
    dstp = jnp.concatenate([dst, jnp.full((EP - E,), N, dst.dtype)]).reshape(
        EP // CH, CH)

    zer128 = jnp.zeros((AGGROW, C), jnp.float32)
    ones128 = jnp.ones((CH, C), jnp.float32)

    deg2 = _sc_deg(dstp, ones128, zer128)

    xs = []
    for p in params['convs']:
        U16 = jnp.tile(p['U'], (1, 4))
        W640 = jnp.concatenate(
            [p['W'], U16, jnp.zeros((C, C - 16), jnp.float32)], axis=1)
        U128 = jnp.concatenate(
            [U16, jnp.zeros((C, C - 16), jnp.float32)], axis=1)
        cpad = jnp.full((1, C), NEG, jnp.float32).at[0, :H].set(p['c'])
        b640 = jnp.zeros((1, 5 * C), jnp.float32).at[0, 4 * C:4 * C + 16].set(
            jnp.tile(p['c'], 4))
        brow = p['b'].reshape(1, C)
        xw, xu = _mm(x, W640, U128, b640)
        agg2 = _sc_edge(xw, xu, srcp, dstp, zer128)
        x = _epilogue(agg2, xw, deg2, cpad, brow)
        xs.append(x[:N])

    wc = params['conv1d_w']
    cb8 = jnp.broadcast_to(params['conv1d_b'][:, None], (8 * C, 8))
    (W0, b0), (W1, b1), (W2, b2) = params['lins']
    w2p = jnp.pad(W2, ((0, 0), (0, C - W2.shape[1])))
    b2p = jnp.pad(b2, (0, C - b2.shape[0])).reshape(1, C)
    out = _head(xs[0], xs[1], xs[2],
                wc[:, 0:C], wc[:, C:2 * C], wc[:, 2 * C:3 * C], cb8,
                W0[:8 * C], W0[8 * C:], b0.reshape(1, 512),
                W1, b1.reshape(1, 256), w2p, b2p)
    return out[:, :10]
